# baseline (speedup 1.0000x reference)
import jax
import jax.numpy as jnp
from jax.experimental import pallas as pl


def _rq_body(x_ref, cb_ref, recon_ref, codes_ref):
    x = x_ref[...]
    L, K, D = cb_ref.shape
    bm = x.shape[0]
    iota = jax.lax.broadcasted_iota(jnp.int32, (bm, K), 1)
    ones = jnp.ones((1, D), dtype=jnp.float32)

    r = x
    recon = jnp.zeros_like(x)
    for l in range(L):
        cb = cb_ref[l]
        c1 = cb.astype(jnp.bfloat16)
        r1 = cb - c1.astype(jnp.float32)
        c2 = r1.astype(jnp.bfloat16)
        c3 = (r1 - c2.astype(jnp.float32)).astype(jnp.bfloat16)
        packed = jnp.concatenate([c1, c2, c3], axis=1)

        a2 = jnp.sum(r * r, axis=1, keepdims=True)
        b2 = jax.lax.dot_general(
            ones, cb * cb, (((1,), (1,)), ((), ())),
            precision=jax.lax.Precision.HIGHEST,
            preferred_element_type=jnp.float32)
        ab = jax.lax.dot_general(
            r, cb, (((1,), (1,)), ((), ())),
            preferred_element_type=jnp.float32)
        d2 = jnp.maximum(a2 + b2 - 2.0 * ab, 0.0)
        m = jnp.min(d2, axis=1, keepdims=True)
        idx = jnp.min(jnp.where(d2 <= m, iota, K), axis=1,
                      keepdims=True)
        codes_ref[:, l:l + 1] = idx
        onehot = (iota == idx).astype(jnp.bfloat16)
        parts = jax.lax.dot_general(
            onehot, packed, (((1,), (0,)), ((), ())),
            preferred_element_type=jnp.float32)
        sel = (parts[:, :D] + parts[:, D:2 * D]) + parts[:, 2 * D:]
        recon = recon + sel
        r = r - sel
    recon_ref[...] = recon


def kernel(x, codebooks):
    B, D = x.shape
    L, K, _ = codebooks.shape
    bm = 1024
    recon, codes_pad = pl.pallas_call(
        _rq_body,
        grid=(B // bm,),
        in_specs=[
            pl.BlockSpec((bm, D), lambda i: (i, 0)),
            pl.BlockSpec((L, K, D), lambda i: (0, 0, 0)),
        ],
        out_specs=[
            pl.BlockSpec((bm, D), lambda i: (i, 0)),
            pl.BlockSpec((bm, 8), lambda i: (i, 0)),
        ],
        out_shape=[
            jax.ShapeDtypeStruct((B, D), jnp.float32),
            jax.ShapeDtypeStruct((B, 8), jnp.int32),
        ],
    )(x, codebooks)
    return recon, codes_pad[:, :L]


# speedup vs baseline: 1.1818x; 1.1818x over previous
import jax
import jax.numpy as jnp
from jax.experimental import pallas as pl
from jax.experimental.pallas import tpu as pltpu


def _rq_body(x_ref, cb_ref, recon_ref, codes_ref, packed_ref, b2_ref):
    x = x_ref[...]
    L, K, D = cb_ref.shape
    bm = x.shape[0]
    iota = jax.lax.broadcasted_iota(jnp.int32, (bm, K), 1)
    ones = jnp.ones((1, D), dtype=jnp.float32)

    @pl.when(pl.program_id(0) == 0)
    def _prep():
        for l in range(L):
            cb = cb_ref[l]
            c1 = cb.astype(jnp.bfloat16)
            r1 = cb - c1.astype(jnp.float32)
            c2 = r1.astype(jnp.bfloat16)
            c3 = (r1 - c2.astype(jnp.float32)).astype(jnp.bfloat16)
            packed_ref[l] = jnp.concatenate([c1, c2, c3], axis=1)
            b2_ref[l] = jax.lax.dot_general(
                ones, cb * cb, (((1,), (1,)), ((), ())),
                precision=jax.lax.Precision.HIGHEST,
                preferred_element_type=jnp.float32)

    r = x
    recon = jnp.zeros_like(x)
    for l in range(L):
        cb = cb_ref[l]
        packed = packed_ref[l]
        a2 = jnp.sum(r * r, axis=1, keepdims=True)
        b2 = b2_ref[l]
        ab = jax.lax.dot_general(
            r, cb, (((1,), (1,)), ((), ())),
            preferred_element_type=jnp.float32)
        d2 = jnp.maximum(a2 + b2 - 2.0 * ab, 0.0)
        m = jnp.min(d2, axis=1, keepdims=True)
        idx = jnp.min(jnp.where(d2 <= m, iota, K), axis=1,
                      keepdims=True)
        codes_ref[:, l:l + 1] = idx
        onehot = (iota == idx).astype(jnp.bfloat16)
        parts = jax.lax.dot_general(
            onehot, packed, (((1,), (0,)), ((), ())),
            preferred_element_type=jnp.float32)
        sel = (parts[:, :D] + parts[:, D:2 * D]) + parts[:, 2 * D:]
        recon = recon + sel
        r = r - sel
    recon_ref[...] = recon


def kernel(x, codebooks):
    B, D = x.shape
    L, K, _ = codebooks.shape
    bm = 1024
    recon, codes_pad = pl.pallas_call(
        _rq_body,
        grid=(B // bm,),
        in_specs=[
            pl.BlockSpec((bm, D), lambda i: (i, 0)),
            pl.BlockSpec((L, K, D), lambda i: (0, 0, 0)),
        ],
        out_specs=[
            pl.BlockSpec((bm, D), lambda i: (i, 0)),
            pl.BlockSpec((bm, 8), lambda i: (i, 0)),
        ],
        out_shape=[
            jax.ShapeDtypeStruct((B, D), jnp.float32),
            jax.ShapeDtypeStruct((B, 8), jnp.int32),
        ],
        scratch_shapes=[
            pltpu.VMEM((L, K, 3 * D), jnp.bfloat16),
            pltpu.VMEM((L, 1, K), jnp.float32),
        ],
    )(x, codebooks)
    return recon, codes_pad[:, :L]


# speedup vs baseline: 1.2584x; 1.0648x over previous
import jax
import jax.numpy as jnp
from jax.experimental import pallas as pl
from jax.experimental.pallas import tpu as pltpu


def _rq_body(x_ref, cb_ref, recon_ref, codes_ref, packed_ref, b2_ref):
    x = x_ref[...]
    L, K, D = cb_ref.shape
    bm = x.shape[0]
    iota = jax.lax.broadcasted_iota(jnp.int32, (bm, K), 1)
    ones = jnp.ones((1, D), dtype=jnp.float32)

    @pl.when(pl.program_id(0) == 0)
    def _prep():
        for l in range(L):
            cb = cb_ref[l]
            c1 = cb.astype(jnp.bfloat16)
            r1 = cb - c1.astype(jnp.float32)
            c2 = r1.astype(jnp.bfloat16)
            c3 = (r1 - c2.astype(jnp.float32)).astype(jnp.bfloat16)
            packed_ref[l] = jnp.concatenate([c1, c2, c3], axis=1)
            b2_ref[l] = jax.lax.dot_general(
                ones, cb * cb, (((1,), (1,)), ((), ())),
                precision=jax.lax.Precision.HIGHEST,
                preferred_element_type=jnp.float32)

    r = x
    recon = jnp.zeros_like(x)
    for l in range(L):
        cb = cb_ref[l]
        packed = packed_ref[l]
        a2 = jnp.sum(r * r, axis=1, keepdims=True)
        b2 = b2_ref[l]
        ab = jax.lax.dot_general(
            r, cb, (((1,), (1,)), ((), ())),
            preferred_element_type=jnp.float32)
        d2 = jnp.maximum(a2 + b2 - 2.0 * ab, 0.0)
        idx = jnp.argmin(d2, axis=1, keepdims=True)
        codes_ref[:, l:l + 1] = idx
        onehot = (iota == idx).astype(jnp.bfloat16)
        parts = jax.lax.dot_general(
            onehot, packed, (((1,), (0,)), ((), ())),
            preferred_element_type=jnp.float32)
        sel = (parts[:, :D] + parts[:, D:2 * D]) + parts[:, 2 * D:]
        recon = recon + sel
        r = r - sel
    recon_ref[...] = recon


def kernel(x, codebooks):
    B, D = x.shape
    L, K, _ = codebooks.shape
    bm = 1024
    recon, codes_pad = pl.pallas_call(
        _rq_body,
        grid=(B // bm,),
        in_specs=[
            pl.BlockSpec((bm, D), lambda i: (i, 0)),
            pl.BlockSpec((L, K, D), lambda i: (0, 0, 0)),
        ],
        out_specs=[
            pl.BlockSpec((bm, D), lambda i: (i, 0)),
            pl.BlockSpec((bm, 8), lambda i: (i, 0)),
        ],
        out_shape=[
            jax.ShapeDtypeStruct((B, D), jnp.float32),
            jax.ShapeDtypeStruct((B, 8), jnp.int32),
        ],
        scratch_shapes=[
            pltpu.VMEM((L, K, 3 * D), jnp.bfloat16),
            pltpu.VMEM((L, 1, K), jnp.float32),
        ],
    )(x, codebooks)
    return recon, codes_pad[:, :L]


# speedup vs baseline: 1.3335x; 1.0597x over previous
import jax
import jax.numpy as jnp
from jax.experimental import pallas as pl
from jax.experimental.pallas import tpu as pltpu


def _rq_body(x_ref, cb_ref, recon_ref, codes_ref, packed_ref, b2_ref):
    x = x_ref[...]
    L, K, D = cb_ref.shape
    bm = x.shape[0]
    iota = jax.lax.broadcasted_iota(jnp.int32, (bm, K), 1)
    ones = jnp.ones((1, D), dtype=jnp.float32)

    @pl.when(pl.program_id(0) == 0)
    def _prep():
        for l in range(L):
            cb = cb_ref[l]
            c1 = cb.astype(jnp.bfloat16)
            r1 = cb - c1.astype(jnp.float32)
            c2 = r1.astype(jnp.bfloat16)
            c3 = (r1 - c2.astype(jnp.float32)).astype(jnp.bfloat16)
            packed_ref[l] = jnp.concatenate([c1, c2, c3], axis=1)
            b2_ref[l] = jax.lax.dot_general(
                ones, cb * cb, (((1,), (1,)), ((), ())),
                precision=jax.lax.Precision.HIGHEST,
                preferred_element_type=jnp.float32)

    r = x
    recon = jnp.zeros_like(x)
    for l in range(L):
        cb = cb_ref[l]
        packed = packed_ref[l]
        a2 = jnp.sum(r * r, axis=1, keepdims=True)
        b2 = b2_ref[l]
        ab = jax.lax.dot_general(
            r, cb, (((1,), (1,)), ((), ())),
            preferred_element_type=jnp.float32)
        d2 = jnp.maximum(a2 + b2 - 2.0 * ab, 0.0)
        idx = jnp.argmin(d2, axis=1, keepdims=True)
        codes_ref[:, l:l + 1] = idx
        onehot = (iota == idx).astype(jnp.bfloat16)
        parts = jax.lax.dot_general(
            onehot, packed, (((1,), (0,)), ((), ())),
            preferred_element_type=jnp.float32)
        sel = (parts[:, :D] + parts[:, D:2 * D]) + parts[:, 2 * D:]
        recon = recon + sel
        r = r - sel
    recon_ref[...] = recon


def kernel(x, codebooks):
    B, D = x.shape
    L, K, _ = codebooks.shape
    bm = 2048
    recon, codes_pad = pl.pallas_call(
        _rq_body,
        grid=(B // bm,),
        in_specs=[
            pl.BlockSpec((bm, D), lambda i: (i, 0)),
            pl.BlockSpec((L, K, D), lambda i: (0, 0, 0)),
        ],
        out_specs=[
            pl.BlockSpec((bm, D), lambda i: (i, 0)),
            pl.BlockSpec((bm, 8), lambda i: (i, 0)),
        ],
        out_shape=[
            jax.ShapeDtypeStruct((B, D), jnp.float32),
            jax.ShapeDtypeStruct((B, 8), jnp.int32),
        ],
        scratch_shapes=[
            pltpu.VMEM((L, K, 3 * D), jnp.bfloat16),
            pltpu.VMEM((L, 1, K), jnp.float32),
        ],
    )(x, codebooks)
    return recon, codes_pad[:, :L]


# speedup vs baseline: 1.3770x; 1.0326x over previous
import jax
import jax.numpy as jnp
from jax.experimental import pallas as pl
from jax.experimental.pallas import tpu as pltpu


def _rq_body(x_ref, cb_ref, recon_ref, codes_ref, packed_ref, b2_ref):
    x = x_ref[...]
    L, K, D = cb_ref.shape
    bm = x.shape[0]
    iota = jax.lax.broadcasted_iota(jnp.int32, (bm, K), 1)
    ones = jnp.ones((1, D), dtype=jnp.float32)

    @pl.when(pl.program_id(0) == 0)
    def _prep():
        for l in range(L):
            cb = cb_ref[l]
            c1 = cb.astype(jnp.bfloat16)
            r1 = cb - c1.astype(jnp.float32)
            c2 = r1.astype(jnp.bfloat16)
            c3 = (r1 - c2.astype(jnp.float32)).astype(jnp.bfloat16)
            packed_ref[l] = jnp.concatenate([c1, c2, c3], axis=1)
            b2_ref[l] = jax.lax.dot_general(
                ones, cb * cb, (((1,), (1,)), ((), ())),
                precision=jax.lax.Precision.HIGHEST,
                preferred_element_type=jnp.float32)

    r = x
    recon = jnp.zeros_like(x)
    for l in range(L):
        cb = cb_ref[l]
        packed = packed_ref[l]
        a2 = jnp.sum(r * r, axis=1, keepdims=True)
        b2 = b2_ref[l]
        ab = jax.lax.dot_general(
            r, cb, (((1,), (1,)), ((), ())),
            preferred_element_type=jnp.float32)
        d2 = jnp.maximum(a2 + b2 - 2.0 * ab, 0.0)
        idx = jnp.argmin(d2, axis=1, keepdims=True)
        codes_ref[:, l:l + 1] = idx
        onehot = (iota == idx).astype(jnp.bfloat16)
        parts = jax.lax.dot_general(
            onehot, packed, (((1,), (0,)), ((), ())),
            preferred_element_type=jnp.float32)
        sel = (parts[:, :D] + parts[:, D:2 * D]) + parts[:, 2 * D:]
        recon = recon + sel
        r = r - sel
    recon_ref[...] = recon


def kernel(x, codebooks):
    B, D = x.shape
    L, K, _ = codebooks.shape
    bm = 4096
    recon, codes_pad = pl.pallas_call(
        _rq_body,
        grid=(B // bm,),
        in_specs=[
            pl.BlockSpec((bm, D), lambda i: (i, 0)),
            pl.BlockSpec((L, K, D), lambda i: (0, 0, 0)),
        ],
        out_specs=[
            pl.BlockSpec((bm, D), lambda i: (i, 0)),
            pl.BlockSpec((bm, 8), lambda i: (i, 0)),
        ],
        out_shape=[
            jax.ShapeDtypeStruct((B, D), jnp.float32),
            jax.ShapeDtypeStruct((B, 8), jnp.int32),
        ],
        scratch_shapes=[
            pltpu.VMEM((L, K, 3 * D), jnp.bfloat16),
            pltpu.VMEM((L, 1, K), jnp.float32),
        ],
    )(x, codebooks)
    return recon, codes_pad[:, :L]


# speedup vs baseline: 1.4119x; 1.0253x over previous
import jax
import jax.numpy as jnp
from jax.experimental import pallas as pl
from jax.experimental.pallas import tpu as pltpu


def _rq_body(x_ref, cb_ref, recon_ref, codes_ref, packed_ref, b2_ref,
             cm2_ref):
    x = x_ref[...]
    L, K, D = cb_ref.shape
    bm = x.shape[0]
    iota = jax.lax.broadcasted_iota(jnp.int32, (bm, K), 1)
    ones = jnp.ones((1, D), dtype=jnp.float32)

    @pl.when(pl.program_id(0) == 0)
    def _prep():
        for l in range(L):
            cb = cb_ref[l]
            c1 = cb.astype(jnp.bfloat16)
            r1 = cb - c1.astype(jnp.float32)
            c2 = r1.astype(jnp.bfloat16)
            c3 = (r1 - c2.astype(jnp.float32)).astype(jnp.bfloat16)
            packed_ref[l] = jnp.concatenate([c1, c2, c3], axis=1)
            b2_ref[l] = jax.lax.dot_general(
                ones, cb * cb, (((1,), (1,)), ((), ())),
                precision=jax.lax.Precision.HIGHEST,
                preferred_element_type=jnp.float32)
            cm2_ref[l] = -2.0 * cb

    r = x
    recon = jnp.zeros_like(x)
    for l in range(L):
        packed = packed_ref[l]
        a2 = jnp.sum(r * r, axis=1, keepdims=True)
        b2 = b2_ref[l]
        ab2 = jax.lax.dot_general(
            r, cm2_ref[l], (((1,), (1,)), ((), ())),
            preferred_element_type=jnp.float32)
        d2 = (a2 + b2) + ab2
        idx = jnp.argmin(d2, axis=1, keepdims=True)
        codes_ref[:, l:l + 1] = idx
        onehot = (iota == idx).astype(jnp.bfloat16)
        parts = jax.lax.dot_general(
            onehot, packed, (((1,), (0,)), ((), ())),
            preferred_element_type=jnp.float32)
        sel = (parts[:, :D] + parts[:, D:2 * D]) + parts[:, 2 * D:]
        recon = recon + sel
        r = r - sel
    recon_ref[...] = recon


def kernel(x, codebooks):
    B, D = x.shape
    L, K, _ = codebooks.shape
    bm = 4096
    recon, codes_pad = pl.pallas_call(
        _rq_body,
        grid=(B // bm,),
        in_specs=[
            pl.BlockSpec((bm, D), lambda i: (i, 0)),
            pl.BlockSpec((L, K, D), lambda i: (0, 0, 0)),
        ],
        out_specs=[
            pl.BlockSpec((bm, D), lambda i: (i, 0)),
            pl.BlockSpec((bm, 8), lambda i: (i, 0)),
        ],
        out_shape=[
            jax.ShapeDtypeStruct((B, D), jnp.float32),
            jax.ShapeDtypeStruct((B, 8), jnp.int32),
        ],
        scratch_shapes=[
            pltpu.VMEM((L, K, 3 * D), jnp.bfloat16),
            pltpu.VMEM((L, 1, K), jnp.float32),
            pltpu.VMEM((L, K, D), jnp.float32),
        ],
    )(x, codebooks)
    return recon, codes_pad[:, :L]
